# Initial kernel scaffold; baseline (speedup 1.0000x reference)
import jax
import jax.numpy as jnp
from jax.experimental import pallas as pl


def kernel(x):
    raise NotImplementedError("write your pallas kernel here")


# speedup vs baseline: 3.4102x; 3.4102x over previous
import jax
import jax.numpy as jnp
from jax.experimental import pallas as pl

ROWS = 128
COLS = 32768
K = 64
BLK = 8


def _body(x_ref, o_ref):
    x = x_ref[...]
    b = jax.lax.bitcast_convert_type(x, jnp.int32)
    v = jnp.where(b >= 0, b, b ^ jnp.int32(0x7FFFFFFF))

    imin = jnp.int32(-2147483648)
    imax = jnp.int32(2147483647)
    lo = jnp.full((BLK, 1), imin, jnp.int32)
    hi = jnp.full((BLK, 1), imax, jnp.int32)

    def step(_, carry):
        lo, hi = carry
        mid = (lo & hi) + ((lo ^ hi) >> 1)
        c = jnp.sum((v >= mid).astype(jnp.int32), axis=1, keepdims=True)
        ge = c >= K
        lo = jnp.where(ge, mid, lo)
        hi = jnp.where(ge, hi, mid)
        return lo, hi

    lo, hi = jax.lax.fori_loop(0, 32, step, (lo, hi), unroll=True)
    t = lo

    gt = v > t
    eq = v == t
    c_gt = jnp.sum(gt.astype(jnp.int32), axis=1, keepdims=True)
    k_eq = K - c_gt

    idx = jax.lax.broadcasted_iota(jnp.int32, (BLK, COLS), 1)

    lo2 = jnp.full((BLK, 1), -1, jnp.int32)
    hi2 = jnp.full((BLK, 1), COLS - 1, jnp.int32)

    def step2(_, carry):
        lo2, hi2 = carry
        mid = (lo2 + hi2) >> 1
        c = jnp.sum((eq & (idx <= mid)).astype(jnp.int32), axis=1, keepdims=True)
        ge = c >= k_eq
        hi2 = jnp.where(ge, mid, hi2)
        lo2 = jnp.where(ge, lo2, mid)
        return lo2, hi2

    lo2, hi2 = jax.lax.fori_loop(0, 15, step2, (lo2, hi2), unroll=True)
    istar = hi2

    mask = gt | (eq & (idx <= istar))
    o_ref[...] = jnp.where(mask, jnp.float32(-1000.0), x)


def kernel(x):
    return pl.pallas_call(
        _body,
        grid=(ROWS // BLK,),
        in_specs=[pl.BlockSpec((BLK, COLS), lambda i: (i, 0))],
        out_specs=pl.BlockSpec((BLK, COLS), lambda i: (i, 0)),
        out_shape=jax.ShapeDtypeStruct((ROWS, COLS), jnp.float32),
    )(x)


# speedup vs baseline: 4.1638x; 1.2210x over previous
import jax
import jax.numpy as jnp
from jax.experimental import pallas as pl

ROWS = 128
COLS = 32768
K = 64
BLK = 8


def _body(x_ref, o_ref):
    x = x_ref[...]
    b = jax.lax.bitcast_convert_type(x, jnp.int32)
    v = jnp.where(b >= 0, b, b ^ jnp.int32(0x7FFFFFFF))

    imin = jnp.int32(-2147483648)
    imax = jnp.int32(2147483647)

    segmax = jnp.max(v.reshape(BLK, COLS // 128, 128), axis=1)
    lo0 = jnp.full((BLK, 1), imin, jnp.int32)
    hi0 = jnp.full((BLK, 1), imax, jnp.int32)

    def step0(_, carry):
        lo0, hi0 = carry
        mid = (lo0 & hi0) + ((lo0 ^ hi0) >> 1)
        c = jnp.sum((segmax >= mid).astype(jnp.int32), axis=1, keepdims=True)
        ge = c >= K
        return jnp.where(ge, mid, lo0), jnp.where(ge, hi0, mid)

    lo0, hi0 = jax.lax.fori_loop(0, 32, step0, (lo0, hi0), unroll=True)
    t0 = lo0
    rowmax = jnp.max(segmax, axis=1, keepdims=True)

    lo = t0
    hi = rowmax + 1
    c_hi = jnp.zeros((BLK, 1), jnp.int32)

    def cond(carry):
        lo, hi, _ = carry
        return jnp.any((hi - lo) > 1)

    def step(carry):
        lo, hi, c_hi = carry
        mid = (lo & hi) + ((lo ^ hi) >> 1)
        c = jnp.sum((v >= mid).astype(jnp.int32), axis=1, keepdims=True)
        ge = c >= K
        return (jnp.where(ge, mid, lo), jnp.where(ge, hi, mid),
                jnp.where(ge, c_hi, c))

    lo, hi, c_hi = jax.lax.while_loop(cond, step, (lo, hi, c_hi))
    t = lo
    c_gt = c_hi

    idx = jax.lax.broadcasted_iota(jnp.int32, (BLK, COLS), 1)
    eqidx = jnp.where(v == t, idx, jnp.int32(COLS))
    rem = K - c_gt
    istar = jnp.full((BLK, 1), -1, jnp.int32)

    def cond2(carry):
        _, rem, _ = carry
        return jnp.any(rem > 0)

    def step2(carry):
        eqidx, rem, istar = carry
        m = jnp.min(eqidx, axis=1, keepdims=True)
        active = rem > 0
        istar = jnp.where(active, m, istar)
        eqidx = jnp.where(active & (eqidx == m), jnp.int32(COLS), eqidx)
        rem = rem - active.astype(jnp.int32)
        return eqidx, rem, istar

    _, _, istar = jax.lax.while_loop(cond2, step2, (eqidx, rem, istar))

    mask = (v > t) | ((v == t) & (idx <= istar))
    o_ref[...] = jnp.where(mask, jnp.float32(-1000.0), x)


def kernel(x):
    return pl.pallas_call(
        _body,
        grid=(ROWS // BLK,),
        in_specs=[pl.BlockSpec((BLK, COLS), lambda i: (i, 0))],
        out_specs=pl.BlockSpec((BLK, COLS), lambda i: (i, 0)),
        out_shape=jax.ShapeDtypeStruct((ROWS, COLS), jnp.float32),
    )(x)


# speedup vs baseline: 4.5037x; 1.0816x over previous
import jax
import jax.numpy as jnp
from jax.experimental import pallas as pl

ROWS = 128
COLS = 32768
K = 64
BLK = 8


def _body(x_ref, o_ref):
    x = x_ref[...]
    b = jax.lax.bitcast_convert_type(x, jnp.int32)
    v = jnp.where(b >= 0, b, b ^ jnp.int32(0x7FFFFFFF))

    imin = jnp.int32(-2147483648)
    imax = jnp.int32(2147483647)

    segmax = jnp.max(v.reshape(BLK, COLS // 128, 128), axis=1)
    lo0 = jnp.full((BLK, 1), imin, jnp.int32)
    hi0 = jnp.full((BLK, 1), imax, jnp.int32)

    def step0(_, carry):
        lo0, hi0 = carry
        mid = (lo0 & hi0) + ((lo0 ^ hi0) >> 1)
        c = jnp.sum((segmax >= mid).astype(jnp.int32), axis=1, keepdims=True)
        ge = c >= K
        return jnp.where(ge, mid, lo0), jnp.where(ge, hi0, mid)

    lo0, hi0 = jax.lax.fori_loop(0, 32, step0, (lo0, hi0), unroll=True)
    t0 = lo0
    rowmax = jnp.max(segmax, axis=1, keepdims=True)

    lo = t0
    hi = rowmax + 1
    c_hi = jnp.zeros((BLK, 1), jnp.int32)

    def cond(carry):
        lo, hi, _ = carry
        return jnp.any((hi - lo) > 1)

    def step(carry):
        lo, hi, c_hi = carry
        stepw = jnp.maximum((hi >> 3) - (lo >> 3), 1)
        ms = [lo + stepw * i for i in range(1, 8)]
        cs = [jnp.sum((v >= m).astype(jnp.int32), axis=1, keepdims=True)
              for m in ms]
        ges = [c >= K for c in cs]
        for m, g in zip(ms, ges):
            lo = jnp.where(g, m, lo)
        for m, g, c in zip(ms[::-1], ges[::-1], cs[::-1]):
            hi = jnp.where(g, hi, m)
            c_hi = jnp.where(g, c_hi, c)
        return lo, hi, c_hi

    lo, hi, c_hi = jax.lax.while_loop(cond, step, (lo, hi, c_hi))
    t = lo
    c_gt = c_hi

    idx = jax.lax.broadcasted_iota(jnp.int32, (BLK, COLS), 1)
    eqidx = jnp.where(v == t, idx, jnp.int32(COLS))
    rem = K - c_gt
    istar = jnp.full((BLK, 1), -1, jnp.int32)

    def cond2(carry):
        _, rem, _ = carry
        return jnp.any(rem > 0)

    def step2(carry):
        eqidx, rem, istar = carry
        m = jnp.min(eqidx, axis=1, keepdims=True)
        active = rem > 0
        istar = jnp.where(active, m, istar)
        eqidx = jnp.where(active & (eqidx == m), jnp.int32(COLS), eqidx)
        rem = rem - active.astype(jnp.int32)
        return eqidx, rem, istar

    _, _, istar = jax.lax.while_loop(cond2, step2, (eqidx, rem, istar))

    mask = (v > t) | ((v == t) & (idx <= istar))
    o_ref[...] = jnp.where(mask, jnp.float32(-1000.0), x)


def kernel(x):
    return pl.pallas_call(
        _body,
        grid=(ROWS // BLK,),
        in_specs=[pl.BlockSpec((BLK, COLS), lambda i: (i, 0))],
        out_specs=pl.BlockSpec((BLK, COLS), lambda i: (i, 0)),
        out_shape=jax.ShapeDtypeStruct((ROWS, COLS), jnp.float32),
    )(x)


# speedup vs baseline: 6.2805x; 1.3945x over previous
import jax
import jax.numpy as jnp
from jax.experimental import pallas as pl

ROWS = 128
COLS = 32768
K = 64
BLK = 32


def _body(x_ref, o_ref):
    x = x_ref[...]
    b = jax.lax.bitcast_convert_type(x, jnp.int32)
    v = jnp.where(b >= 0, b, b ^ jnp.int32(0x7FFFFFFF))

    imin = jnp.int32(-2147483648)
    imax = jnp.int32(2147483647)

    segmax = jnp.max(v.reshape(BLK, COLS // 128, 128), axis=1)
    lo0 = jnp.full((BLK, 1), imin, jnp.int32)
    hi0 = jnp.full((BLK, 1), imax, jnp.int32)

    def step0(_, carry):
        lo0, hi0 = carry
        mid = (lo0 & hi0) + ((lo0 ^ hi0) >> 1)
        c = jnp.sum((segmax >= mid).astype(jnp.int32), axis=1, keepdims=True)
        ge = c >= K
        return jnp.where(ge, mid, lo0), jnp.where(ge, hi0, mid)

    lo0, hi0 = jax.lax.fori_loop(0, 32, step0, (lo0, hi0), unroll=True)
    t0 = lo0
    rowmax = jnp.max(segmax, axis=1, keepdims=True)

    lo = t0
    hi = rowmax + 1
    c_hi = jnp.zeros((BLK, 1), jnp.int32)

    def cond(carry):
        lo, hi, _ = carry
        return jnp.any((hi - lo) > 1)

    def step(carry):
        lo, hi, c_hi = carry
        stepw = jnp.maximum((hi >> 3) - (lo >> 3), 1)
        ms = [lo + stepw * i for i in range(1, 8)]
        cs = [jnp.sum((v >= m).astype(jnp.int32), axis=1, keepdims=True)
              for m in ms]
        ges = [c >= K for c in cs]
        for m, g in zip(ms, ges):
            lo = jnp.where(g, m, lo)
        for m, g, c in zip(ms[::-1], ges[::-1], cs[::-1]):
            hi = jnp.where(g, hi, m)
            c_hi = jnp.where(g, c_hi, c)
        return lo, hi, c_hi

    lo, hi, c_hi = jax.lax.while_loop(cond, step, (lo, hi, c_hi))
    t = lo
    c_gt = c_hi

    idx = jax.lax.broadcasted_iota(jnp.int32, (BLK, COLS), 1)
    eqidx = jnp.where(v == t, idx, jnp.int32(COLS))
    rem = K - c_gt
    istar = jnp.full((BLK, 1), -1, jnp.int32)

    def cond2(carry):
        _, rem, _ = carry
        return jnp.any(rem > 0)

    def step2(carry):
        eqidx, rem, istar = carry
        m = jnp.min(eqidx, axis=1, keepdims=True)
        active = rem > 0
        istar = jnp.where(active, m, istar)
        eqidx = jnp.where(active & (eqidx == m), jnp.int32(COLS), eqidx)
        rem = rem - active.astype(jnp.int32)
        return eqidx, rem, istar

    _, _, istar = jax.lax.while_loop(cond2, step2, (eqidx, rem, istar))

    mask = (v > t) | ((v == t) & (idx <= istar))
    o_ref[...] = jnp.where(mask, jnp.float32(-1000.0), x)


def kernel(x):
    return pl.pallas_call(
        _body,
        grid=(ROWS // BLK,),
        in_specs=[pl.BlockSpec((BLK, COLS), lambda i: (i, 0))],
        out_specs=pl.BlockSpec((BLK, COLS), lambda i: (i, 0)),
        out_shape=jax.ShapeDtypeStruct((ROWS, COLS), jnp.float32),
    )(x)
